# baseline (speedup 1.0000x reference)
import functools

import jax
import jax.numpy as jnp
from jax import lax
from jax.experimental import pallas as pl
from jax.experimental.pallas import tpu as pltpu
from jax.experimental.pallas import tpu_sc as plsc

EMB = 128
VP = 1024
TP = 1008
TSPL = 2
NC, NS, L = 2, 16, 16
NW = NC * NS
NSEG = 8
KBLK = 8
TBLK = 4


def _tc_body(a_full, a_blk, tgt_ref, ctxT_ref, gram_ref, idx_ref):
    t = pl.program_id(0)
    k = pl.program_id(1)
    g = lax.dot_general(
        a_full[...], a_blk[...], (((1,), (1,)), ((), ())),
        preferred_element_type=jnp.float32)
    gram_ref[...] = g.reshape(1, TP // TSPL, VP // KBLK)

    @pl.when((k == 0) & (t == 0))
    def _():
        ctx = ctxT_ref[...]
        idx_ref[pl.ds(0, ctxT_ref.shape[0]), :] = (
            (ctx >> 7) * (TP * (VP // KBLK))
            + tgt_ref[...] * (VP // KBLK) + (ctx & (VP // KBLK - 1)))


def _tc_call(emb, target_row, contextT):
    C, B = contextT.shape
    CP = (C + 7) // 8 * 8
    return pl.pallas_call(
        _tc_body,
        grid=(TSPL, KBLK),
        in_specs=[
            pl.BlockSpec((TP // TSPL, EMB), lambda t, k: (t, 0)),
            pl.BlockSpec((VP // KBLK, EMB), lambda t, k: (k, 0)),
            pl.BlockSpec((1, B), lambda t, k: (0, 0)),
            pl.BlockSpec((C, B), lambda t, k: (0, 0)),
        ],
        out_specs=[
            pl.BlockSpec((1, TP // TSPL, VP // KBLK), lambda t, k: (k, t, 0)),
            pl.BlockSpec((CP, B), lambda t, k: (0, 0)),
        ],
        out_shape=(jax.ShapeDtypeStruct((KBLK, TP, VP // KBLK), jnp.float32),
                   jax.ShapeDtypeStruct((CP, B), jnp.int32)),
    )(emb, emb, target_row, contextT)


def _make_sc_gather(B, C):
    assert B * C % NW == 0
    PAIRS = B * C // NW

    mesh = plsc.VectorSubcoreMesh(
        core_axis_name="c", subcore_axis_name="s",
        num_cores=NC, num_subcores=NS)

    @functools.partial(
        pl.kernel,
        out_type=jax.ShapeDtypeStruct((B * C,), jnp.float32),
        mesh=mesh,
        scratch_types=[
            pltpu.VMEM((PAIRS,), jnp.int32),
            pltpu.VMEM((PAIRS,), jnp.float32),
            pltpu.SemaphoreType.DMA,
            pltpu.SemaphoreType.DMA,
            pltpu.SemaphoreType.DMA,
        ],
    )
    def sc_gather(gram_hbm, idx_hbm, out_hbm, idx_v, rows_v,
                  sem_i, sem_g, sem_o):
        wid = lax.axis_index("s") * NC + lax.axis_index("c")
        base = wid * PAIRS
        SEG = PAIRS // NSEG
        for s in range(NSEG):
            pltpu.async_copy(idx_hbm.at[pl.ds(base + s * SEG, SEG)],
                             idx_v.at[pl.ds(s * SEG, SEG)], sem_i)
        for s in range(NSEG):
            pltpu.make_async_copy(idx_hbm.at[pl.ds(base + s * SEG, SEG)],
                                  idx_v.at[pl.ds(s * SEG, SEG)], sem_i).wait()
            pltpu.async_copy(gram_hbm.at[idx_v.at[pl.ds(s * SEG, SEG)]],
                             rows_v.at[pl.ds(s * SEG, SEG)], sem_g)
        for s in range(NSEG):
            pltpu.make_async_copy(gram_hbm.at[idx_v.at[pl.ds(s * SEG, SEG)]],
                                  rows_v.at[pl.ds(s * SEG, SEG)], sem_g).wait()
            pltpu.async_copy(rows_v.at[pl.ds(s * SEG, SEG)],
                             out_hbm.at[pl.ds(base + s * SEG, SEG)], sem_o)
        for s in range(NSEG):
            pltpu.make_async_copy(rows_v.at[pl.ds(s * SEG, SEG)],
                                  out_hbm.at[pl.ds(base + s * SEG, SEG)],
                                  sem_o).wait()

    return sc_gather


def kernel(target, context, embd):
    B, = target.shape
    C = context.shape[1]
    CP = (C + 7) // 8 * 8
    g, idxT = _tc_call(embd, target.reshape(1, B), context.T)
    out = _make_sc_gather(B, C)(g.reshape(KBLK * TP * (VP // KBLK)),
                                idxT.reshape(CP * B))
    return out.reshape(C, B).T


# speedup vs baseline: 1.1109x; 1.1109x over previous
import functools

import jax
import jax.numpy as jnp
from jax import lax
from jax.experimental import pallas as pl
from jax.experimental.pallas import tpu as pltpu
from jax.experimental.pallas import tpu_sc as plsc

EMB = 128
VP = 1024
TP = 1008
NC, NS, L = 2, 16, 16
NW = NC * NS
NSEG = 8
KBLK = 8
TBLK = 4


def _tc_body(a_full, a_blk, tgt_ref, ctxT_ref, gram_ref, idx_ref):
    k = pl.program_id(0)
    g = lax.dot_general(
        a_full[...], a_blk[...], (((1,), (1,)), ((), ())),
        preferred_element_type=jnp.float32)
    gram_ref[...] = g.reshape(1, TP, VP // KBLK)

    @pl.when(k == 0)
    def _():
        ctx = ctxT_ref[...]
        idx_ref[pl.ds(0, ctxT_ref.shape[0]), :] = (
            (ctx >> 7) * (TP * (VP // KBLK))
            + tgt_ref[...] * (VP // KBLK) + (ctx & (VP // KBLK - 1)))


def _tc_call(emb, target_row, contextT):
    C, B = contextT.shape
    CP = (C + 7) // 8 * 8
    return pl.pallas_call(
        _tc_body,
        grid=(KBLK,),
        in_specs=[
            pl.BlockSpec((TP, EMB), lambda k: (0, 0)),
            pl.BlockSpec((VP // KBLK, EMB), lambda k: (k, 0)),
            pl.BlockSpec((1, B), lambda k: (0, 0)),
            pl.BlockSpec((C, B), lambda k: (0, 0)),
        ],
        out_specs=[
            pl.BlockSpec((1, TP, VP // KBLK), lambda k: (k, 0, 0)),
            pl.BlockSpec((CP, B), lambda k: (0, 0)),
        ],
        out_shape=(jax.ShapeDtypeStruct((KBLK, TP, VP // KBLK), jnp.float32),
                   jax.ShapeDtypeStruct((CP, B), jnp.int32)),
    )(emb, emb, target_row, contextT)


def _make_sc_gather(B, C):
    assert B * C % NW == 0
    PAIRS = B * C // NW

    mesh = plsc.VectorSubcoreMesh(
        core_axis_name="c", subcore_axis_name="s",
        num_cores=NC, num_subcores=NS)

    @functools.partial(
        pl.kernel,
        out_type=jax.ShapeDtypeStruct((B * C,), jnp.float32),
        mesh=mesh,
        scratch_types=[
            pltpu.VMEM((PAIRS,), jnp.int32),
            pltpu.VMEM((PAIRS,), jnp.float32),
            pltpu.SemaphoreType.DMA,
            pltpu.SemaphoreType.DMA,
            pltpu.SemaphoreType.DMA,
        ],
    )
    def sc_gather(gram_hbm, idx_hbm, out_hbm, idx_v, rows_v,
                  sem_i, sem_g, sem_o):
        wid = lax.axis_index("s") * NC + lax.axis_index("c")
        base = wid * PAIRS
        SEG = PAIRS // NSEG
        for s in range(NSEG):
            pltpu.async_copy(idx_hbm.at[pl.ds(base + s * SEG, SEG)],
                             idx_v.at[pl.ds(s * SEG, SEG)], sem_i)
        for s in range(NSEG):
            pltpu.make_async_copy(idx_hbm.at[pl.ds(base + s * SEG, SEG)],
                                  idx_v.at[pl.ds(s * SEG, SEG)], sem_i).wait()
            pltpu.async_copy(gram_hbm.at[idx_v.at[pl.ds(s * SEG, SEG)]],
                             rows_v.at[pl.ds(s * SEG, SEG)], sem_g)
        for s in range(NSEG):
            pltpu.make_async_copy(gram_hbm.at[idx_v.at[pl.ds(s * SEG, SEG)]],
                                  rows_v.at[pl.ds(s * SEG, SEG)], sem_g).wait()
            pltpu.async_copy(rows_v.at[pl.ds(s * SEG, SEG)],
                             out_hbm.at[pl.ds(base + s * SEG, SEG)], sem_o)
        for s in range(NSEG):
            pltpu.make_async_copy(rows_v.at[pl.ds(s * SEG, SEG)],
                                  out_hbm.at[pl.ds(base + s * SEG, SEG)],
                                  sem_o).wait()

    return sc_gather


def kernel(target, context, embd):
    B, = target.shape
    C = context.shape[1]
    CP = (C + 7) // 8 * 8
    g, idxT = _tc_call(embd, target.reshape(1, B), context.T)
    out = _make_sc_gather(B, C)(g.reshape(KBLK * TP * (VP // KBLK)),
                                idxT.reshape(CP * B))
    return out.reshape(C, B).T


# speedup vs baseline: 1.2627x; 1.1366x over previous
import functools

import jax
import jax.numpy as jnp
from jax import lax
from jax.experimental import pallas as pl
from jax.experimental.pallas import tpu as pltpu
from jax.experimental.pallas import tpu_sc as plsc

EMB = 128
VP = 1024
TP = 1008
NC, NS, L = 2, 16, 16
NW = NC * NS
NSEG = 8
KBLK = 8
TBLK = 4


def _tc_body(a_full, a_blk, tgt_ref, ctxT_ref, gram_ref, idx_ref):
    k = pl.program_id(0)
    g = lax.dot_general(
        a_full[...], a_blk[...], (((1,), (1,)), ((), ())),
        preferred_element_type=jnp.float32)
    gram_ref[...] = g.reshape(1, TP, VP // KBLK)

    @pl.when(k == 0)
    def _():
        ctx = ctxT_ref[...]
        idx_ref[pl.ds(0, ctxT_ref.shape[0]), :] = (
            (ctx >> 7) * (TP * (VP // KBLK))
            + tgt_ref[...] * (VP // KBLK) + (ctx & (VP // KBLK - 1)))


def _tc_call(emb, target_row, contextT):
    C, B = contextT.shape
    CP = (C + 7) // 8 * 8
    return pl.pallas_call(
        _tc_body,
        grid=(KBLK,),
        in_specs=[
            pl.BlockSpec((TP, EMB), lambda k: (0, 0)),
            pl.BlockSpec((VP // KBLK, EMB), lambda k: (k, 0)),
            pl.BlockSpec((1, B), lambda k: (0, 0)),
            pl.BlockSpec((C, B), lambda k: (0, 0)),
        ],
        out_specs=[
            pl.BlockSpec((1, TP, VP // KBLK), lambda k: (k, 0, 0)),
            pl.BlockSpec((CP, B), lambda k: (0, 0)),
        ],
        out_shape=(jax.ShapeDtypeStruct((KBLK, TP, VP // KBLK), jnp.float32),
                   jax.ShapeDtypeStruct((CP, B), jnp.int32)),
    )(emb, emb, target_row, contextT)


def _make_sc_gather(B, C):
    CP = (C + 7) // 8 * 8
    A_FULL = (C // 8) * 8 * B
    A_W = A_FULL // NW
    BL = (C % 8) * 128
    NB_W = (B // 128) // NW
    PAIRS = A_W + NB_W * BL
    assert PAIRS * NW == B * C and A_W % NSEG == 0
    SEG_A = A_W // NSEG

    mesh = plsc.VectorSubcoreMesh(
        core_axis_name="c", subcore_axis_name="s",
        num_cores=NC, num_subcores=NS)

    @functools.partial(
        pl.kernel,
        out_type=jax.ShapeDtypeStruct((CP * B,), jnp.float32),
        mesh=mesh,
        scratch_types=[
            pltpu.VMEM((PAIRS,), jnp.int32),
            pltpu.VMEM((PAIRS,), jnp.float32),
            pltpu.SemaphoreType.DMA,
            pltpu.SemaphoreType.DMA,
            pltpu.SemaphoreType.DMA,
        ],
    )
    def sc_gather(gram_hbm, idx_hbm, out_hbm, idx_v, rows_v,
                  sem_i, sem_g, sem_o):
        wid = lax.axis_index("s") * NC + lax.axis_index("c")
        base_a = wid * A_W
        segs = [(base_a + s * SEG_A, s * SEG_A, SEG_A) for s in range(NSEG)]
        segs += [(A_FULL + (wid * NB_W + j) * (8 * 128), A_W + j * BL, BL)
                 for j in range(NB_W)]
        for ho, lo, sz in segs:
            pltpu.async_copy(idx_hbm.at[pl.ds(ho, sz)],
                             idx_v.at[pl.ds(lo, sz)], sem_i)
        for ho, lo, sz in segs:
            pltpu.make_async_copy(idx_hbm.at[pl.ds(ho, sz)],
                                  idx_v.at[pl.ds(lo, sz)], sem_i).wait()
            pltpu.async_copy(gram_hbm.at[idx_v.at[pl.ds(lo, sz)]],
                             rows_v.at[pl.ds(lo, sz)], sem_g)
        for ho, lo, sz in segs:
            pltpu.make_async_copy(gram_hbm.at[idx_v.at[pl.ds(lo, sz)]],
                                  rows_v.at[pl.ds(lo, sz)], sem_g).wait()
            pltpu.async_copy(rows_v.at[pl.ds(lo, sz)],
                             out_hbm.at[pl.ds(ho, sz)], sem_o)
        for ho, lo, sz in segs:
            pltpu.make_async_copy(rows_v.at[pl.ds(lo, sz)],
                                  out_hbm.at[pl.ds(ho, sz)], sem_o).wait()

    return sc_gather


def kernel(target, context, embd):
    B, = target.shape
    C = context.shape[1]
    CP = (C + 7) // 8 * 8
    TR, BT = CP // 8, B // 128
    g, idxT = _tc_call(embd, target.reshape(1, B), context.T)
    idx_p = idxT.reshape(TR, 8, BT, 128).transpose(0, 2, 1, 3).reshape(CP * B)
    out = _make_sc_gather(B, C)(g.reshape(KBLK * TP * (VP // KBLK)), idx_p)
    return out.reshape(TR, BT, 8, 128).transpose(1, 3, 0, 2).reshape(
        B, CP)[:, :C]


# speedup vs baseline: 1.2692x; 1.0052x over previous
import functools

import jax
import jax.numpy as jnp
from jax import lax
from jax.experimental import pallas as pl
from jax.experimental.pallas import tpu as pltpu
from jax.experimental.pallas import tpu_sc as plsc

EMB = 128
VP = 1024
TP = 1008
NC, NS, L = 2, 16, 16
NW = NC * NS
NSEG = 2
KBLK = 8
TBLK = 4


def _tc_body(a_full, a_blk, tgt_ref, ctxT_ref, gram_ref, idx_ref):
    k = pl.program_id(0)
    g = lax.dot_general(
        a_full[...], a_blk[...], (((1,), (1,)), ((), ())),
        preferred_element_type=jnp.float32)
    gram_ref[...] = g.reshape(1, TP, VP // KBLK)

    @pl.when(k == 0)
    def _():
        ctx = ctxT_ref[...]
        idx_ref[pl.ds(0, ctxT_ref.shape[0]), :] = (
            (ctx >> 7) * (TP * (VP // KBLK))
            + tgt_ref[...] * (VP // KBLK) + (ctx & (VP // KBLK - 1)))


def _tc_call(emb, target_row, contextT):
    C, B = contextT.shape
    CP = (C + 7) // 8 * 8
    return pl.pallas_call(
        _tc_body,
        grid=(KBLK,),
        in_specs=[
            pl.BlockSpec((TP, EMB), lambda k: (0, 0)),
            pl.BlockSpec((VP // KBLK, EMB), lambda k: (k, 0)),
            pl.BlockSpec((1, B), lambda k: (0, 0)),
            pl.BlockSpec((C, B), lambda k: (0, 0)),
        ],
        out_specs=[
            pl.BlockSpec((1, TP, VP // KBLK), lambda k: (k, 0, 0)),
            pl.BlockSpec((CP, B), lambda k: (0, 0)),
        ],
        out_shape=(jax.ShapeDtypeStruct((KBLK, TP, VP // KBLK), jnp.float32),
                   jax.ShapeDtypeStruct((CP, B), jnp.int32)),
    )(emb, emb, target_row, contextT)


def _make_sc_gather(B, C):
    CP = (C + 7) // 8 * 8
    A_FULL = (C // 8) * 8 * B
    A_W = A_FULL // NW
    BL = (C % 8) * 128
    NB_W = (B // 128) // NW
    PAIRS = A_W + NB_W * BL
    assert PAIRS * NW == B * C and A_W % NSEG == 0
    SEG_A = A_W // NSEG

    mesh = plsc.VectorSubcoreMesh(
        core_axis_name="c", subcore_axis_name="s",
        num_cores=NC, num_subcores=NS)

    @functools.partial(
        pl.kernel,
        out_type=jax.ShapeDtypeStruct((CP * B,), jnp.float32),
        mesh=mesh,
        scratch_types=[
            pltpu.VMEM((PAIRS,), jnp.int32),
            pltpu.VMEM((PAIRS,), jnp.float32),
            pltpu.SemaphoreType.DMA,
            pltpu.SemaphoreType.DMA,
            pltpu.SemaphoreType.DMA,
        ],
    )
    def sc_gather(gram_hbm, idx_hbm, out_hbm, idx_v, rows_v,
                  sem_i, sem_g, sem_o):
        wid = lax.axis_index("s") * NC + lax.axis_index("c")
        base_a = wid * A_W
        segs = [(base_a + s * SEG_A, s * SEG_A, SEG_A) for s in range(NSEG)]
        segs += [(A_FULL + (wid * NB_W + j) * (8 * 128), A_W + j * BL, BL)
                 for j in range(NB_W)]
        for ho, lo, sz in segs:
            pltpu.async_copy(idx_hbm.at[pl.ds(ho, sz)],
                             idx_v.at[pl.ds(lo, sz)], sem_i)
        for ho, lo, sz in segs:
            pltpu.make_async_copy(idx_hbm.at[pl.ds(ho, sz)],
                                  idx_v.at[pl.ds(lo, sz)], sem_i).wait()
            pltpu.async_copy(gram_hbm.at[idx_v.at[pl.ds(lo, sz)]],
                             rows_v.at[pl.ds(lo, sz)], sem_g)
        for ho, lo, sz in segs:
            pltpu.make_async_copy(gram_hbm.at[idx_v.at[pl.ds(lo, sz)]],
                                  rows_v.at[pl.ds(lo, sz)], sem_g).wait()
            pltpu.async_copy(rows_v.at[pl.ds(lo, sz)],
                             out_hbm.at[pl.ds(ho, sz)], sem_o)
        for ho, lo, sz in segs:
            pltpu.make_async_copy(rows_v.at[pl.ds(lo, sz)],
                                  out_hbm.at[pl.ds(ho, sz)], sem_o).wait()

    return sc_gather


def kernel(target, context, embd):
    B, = target.shape
    C = context.shape[1]
    CP = (C + 7) // 8 * 8
    TR, BT = CP // 8, B // 128
    g, idxT = _tc_call(embd, target.reshape(1, B), context.T)
    idx_p = idxT.reshape(TR, 8, BT, 128).transpose(0, 2, 1, 3).reshape(CP * B)
    out = _make_sc_gather(B, C)(g.reshape(KBLK * TP * (VP // KBLK)), idx_p)
    return out.reshape(TR, BT, 8, 128).transpose(1, 3, 0, 2).reshape(
        B, CP)[:, :C]
